# baseline (device time: 20922 ns/iter reference)
import functools
import os

import jax
import jax.numpy as jnp
from jax import lax
from jax.experimental import pallas as pl
from jax.experimental.pallas import tpu as pltpu

N_DEV = 32
K_CHUNK = int(os.environ.get("KCHUNK", "1024"))
W_DEPTH = int(os.environ.get("KDEPTH", "3"))

_VARIANT = os.environ.get("KVARIANT", "full")

_sem_signal = getattr(pl, "semaphore_signal", None) or pltpu.semaphore_signal
_sem_wait = getattr(pl, "semaphore_wait", None) or pltpu.semaphore_wait
_CompilerParams = getattr(pltpu, "CompilerParams", None) or pltpu.TPUCompilerParams

def _spiral(n):
    return tuple(sorted(range(n), key=lambda o: (min(o, n - o), o)))


def kernel(x, w_mat, scale_x, scale_w):
    m_total, k_per = x.shape
    k_total, n = w_mat.shape
    m_per = m_total // N_DEV
    comm_dtype = jnp.float8_e5m2
    mxu_dtype = jnp.float8_e5m2
    n_chunks = k_total // K_CHUNK
    blocks_per_chunk = K_CHUNK // k_per
    spiral = _spiral(n_chunks)

    no_comm = _VARIANT in ("local", "nocast", "nowdma", "barrieronly")
    do_barrier = _VARIANT in ("full", "barrieronly")
    do_wdma = _VARIANT != "nowdma"
    do_cast = _VARIANT in ("full", "local", "barrieronly")

    def body(x_hbm, w_hbm, sx_ref, sw_ref, out_ref,
             xf_ref, x8_ref, xg_ref, wbuf_ref, wc_ref,
             send_sems, recv_sems, w_sems, x_sem):
        my = lax.axis_index("i")
        my_chunk = my // blocks_per_chunk

        def chunk_at(t):
            return (my_chunk + spiral[t]) % n_chunks

        if do_barrier:
            barrier = pltpu.get_barrier_semaphore()
            for d in range(1, N_DEV):
                _sem_signal(barrier, inc=1, device_id=((my + d) % N_DEV,),
                            device_id_type=pl.DeviceIdType.MESH)

        x_dma = pltpu.make_async_copy(x_hbm, xf_ref, x_sem)
        x_dma.start()

        def _w_dma(t):
            c = chunk_at(t)
            return pltpu.make_async_copy(
                w_hbm.at[pl.ds(c * K_CHUNK, K_CHUNK), :],
                wbuf_ref.at[t % W_DEPTH],
                w_sems.at[t % W_DEPTH],
            )

        w_dmas = {}
        if do_wdma:
            for t in range(min(W_DEPTH, n_chunks)):
                w_dmas[t] = _w_dma(t)
                w_dmas[t].start()

        x_dma.wait()
        x8_ref[:, :] = xf_ref[:, :].astype(comm_dtype)

        if no_comm:
            xg_ref[my] = x8_ref[pl.ds(my * m_per, m_per), :]
        else:
            pltpu.make_async_copy(
                x8_ref.at[pl.ds(my * m_per, m_per), :],
                xg_ref.at[my],
                recv_sems.at[my],
            ).start()

        sends = []
        if do_barrier:
            _sem_wait(barrier, N_DEV - 1)
        if not no_comm:
            for d in range(1, N_DEV):
                dst = (my + d) % N_DEV
                rdma = pltpu.make_async_remote_copy(
                    src_ref=x8_ref.at[pl.ds(dst * m_per, m_per), :],
                    dst_ref=xg_ref.at[my],
                    send_sem=send_sems.at[d - 1],
                    recv_sem=recv_sems.at[my],
                    device_id=(dst,),
                    device_id_type=pl.DeviceIdType.MESH,
                )
                rdma.start()
                sends.append(rdma)

        s = sx_ref[0] * sw_ref[0]

        for t in range(n_chunks):
            c = chunk_at(t)
            if do_wdma:
                w_dmas[t].wait()
            if do_cast:
                wc_ref[t % 2] = wbuf_ref[t % W_DEPTH].astype(mxu_dtype)
            if do_wdma:
                nt = t + W_DEPTH
                if nt < n_chunks:
                    w_dmas[nt] = _w_dma(nt)
                    w_dmas[nt].start()

            if not no_comm:
                for i in range(blocks_per_chunk):
                    j = c * blocks_per_chunk + i
                    pltpu.make_async_copy(
                        x8_ref.at[pl.ds(0, m_per), :],
                        xg_ref.at[j],
                        recv_sems.at[j],
                    ).wait()

            xb = jnp.concatenate(
                [xg_ref[c * blocks_per_chunk + i] for i in range(blocks_per_chunk)],
                axis=1,
            )
            part = jnp.dot(xb, wc_ref[t % 2], preferred_element_type=jnp.float32)
            if t == 0:
                out_ref[:, :] = part
            else:
                out_ref[:, :] = out_ref[:, :] + part

        out_ref[:, :] = jnp.maximum(out_ref[:, :] * s, 0.0)

        for rdma in sends:
            rdma.wait_send()

    return pl.pallas_call(
        body,
        out_shape=jax.ShapeDtypeStruct((m_per, n), jnp.float32),
        in_specs=[
            pl.BlockSpec(memory_space=pl.ANY),
            pl.BlockSpec(memory_space=pl.ANY),
            pl.BlockSpec(memory_space=pltpu.MemorySpace.SMEM),
            pl.BlockSpec(memory_space=pltpu.MemorySpace.SMEM),
        ],
        out_specs=pl.BlockSpec(memory_space=pltpu.MemorySpace.VMEM),
        scratch_shapes=[
            pltpu.VMEM((m_total, k_per), jnp.float32),
            pltpu.VMEM((m_total, k_per), comm_dtype),
            pltpu.VMEM((N_DEV, m_per, k_per), comm_dtype),
            pltpu.VMEM((W_DEPTH, K_CHUNK, n), jnp.float32),
            pltpu.VMEM((2, K_CHUNK, n), mxu_dtype),
            pltpu.SemaphoreType.DMA((N_DEV - 1,)),
            pltpu.SemaphoreType.DMA((N_DEV,)),
            pltpu.SemaphoreType.DMA((W_DEPTH,)),
            pltpu.SemaphoreType.DMA,
        ],
        compiler_params=(_CompilerParams(collective_id=0) if do_barrier
                         else _CompilerParams()),
    )(x, w_mat, scale_x, scale_w)
